# baseline (speedup 1.0000x reference)
import functools

import jax
import jax.numpy as jnp
from jax import lax
from jax.experimental import pallas as pl
from jax.experimental.pallas import tpu as pltpu
from jax.experimental.pallas import tpu_sc as plsc

L = 16
NC = 2
NS = 16
NW = NC * NS


def _enc_body(t_ref, enc_ref):
    t = t_ref[...]
    norms = jnp.sqrt(jnp.sum(t * t, axis=-1, keepdims=True))
    scale = jnp.where(norms > 1.0, 1.0 / (norms + 1e-7), 1.0)
    enc_ref[...] = t * scale


def _compute_enc(table):
    S, D = table.shape
    return pl.pallas_call(
        _enc_body,
        out_shape=jax.ShapeDtypeStruct((S, D), jnp.float32),
    )(table)


def kernel(x, table):
    B, S, one, D = x.shape
    words_per_row = S * D // L
    rows_per_w = B // NW

    enc = _compute_enc(table)
    enc2 = enc.reshape(words_per_row, L)
    x2 = x.reshape(B * words_per_row, L)

    mesh = plsc.VectorSubcoreMesh(core_axis_name="c", subcore_axis_name="s")

    @functools.partial(
        pl.kernel,
        mesh=mesh,
        out_type=jax.ShapeDtypeStruct((B * words_per_row, L), jnp.float32),
        scratch_types=[
            pltpu.VMEM((words_per_row, L), jnp.float32),
            pltpu.VMEM((words_per_row, L), jnp.float32),
        ],
        compiler_params=pltpu.CompilerParams(use_tc_tiling_on_sc=False),
    )
    def sc_add(x_hbm, enc_hbm, out_hbm, enc_v, buf):
        wid = lax.axis_index("s") * NC + lax.axis_index("c")
        base = wid * rows_per_w

        pltpu.sync_copy(enc_hbm, enc_v)

        def add8(j, carry):
            for k in range(8):
                i = j * 8 + k
                buf[i] = buf[i] + enc_v[i]
            return carry

        def per_row(r, carry):
            off = (base + r) * words_per_row
            pltpu.sync_copy(x_hbm.at[pl.ds(off, words_per_row)], buf)
            lax.fori_loop(0, words_per_row // 8, add8, 0, unroll=False)
            pltpu.sync_copy(buf, out_hbm.at[pl.ds(off, words_per_row)])
            return carry

        lax.fori_loop(0, rows_per_w, per_row, 0, unroll=False)

    out = sc_add(x2, enc2)
    return out.reshape(B, S, one, D)


# speedup vs baseline: 2.7002x; 2.7002x over previous
import jax
import jax.numpy as jnp
from jax.experimental import pallas as pl
from jax.experimental.pallas import tpu as pltpu

B_BLK = 128


def _body(x_ref, t_ref, o_ref, enc_ref):
    @pl.when(pl.program_id(0) == 0)
    def _():
        t = t_ref[...]
        norms = jnp.sqrt(jnp.sum(t * t, axis=-1, keepdims=True))
        scale = jnp.where(norms > 1.0, 1.0 / (norms + 1e-7), 1.0)
        enc_ref[...] = t * scale

    o_ref[...] = x_ref[...] + enc_ref[...]


def kernel(x, table):
    B, S, one, D = x.shape
    x3 = x.reshape(B, S, D)
    grid = (B // B_BLK,)
    out = pl.pallas_call(
        _body,
        grid=grid,
        in_specs=[
            pl.BlockSpec((B_BLK, S, D), lambda i: (i, 0, 0)),
            pl.BlockSpec((S, D), lambda i: (0, 0)),
        ],
        out_specs=pl.BlockSpec((B_BLK, S, D), lambda i: (i, 0, 0)),
        out_shape=jax.ShapeDtypeStruct((B, S, D), x.dtype),
        scratch_shapes=[pltpu.VMEM((S, D), jnp.float32)],
        compiler_params=pltpu.CompilerParams(
            dimension_semantics=("arbitrary",),
            vmem_limit_bytes=64 * 1024 * 1024,
        ),
    )(x3, table)
    return out.reshape(B, S, one, D)
